# Initial kernel scaffold; baseline (speedup 1.0000x reference)
import jax
import jax.numpy as jnp
from jax.experimental import pallas as pl


def kernel(input_ids, wte):
    raise NotImplementedError("write your pallas kernel here")


# speedup vs baseline: 1.4132x; 1.4132x over previous
import functools

import jax
import jax.numpy as jnp
from jax import lax
from jax.experimental import pallas as pl
from jax.experimental.pallas import tpu as pltpu
from jax.experimental.pallas import tpu_sc as plsc

VOCAB = 50304
N_EMBD = 1024
B_TOTAL = 4 * 4096
NUM_CORES = 2
NUM_SUBCORES = 16
NUM_WORKERS = NUM_CORES * NUM_SUBCORES
B_PER_W = B_TOTAL // NUM_WORKERS
CHUNK = 32
N_CHUNKS = B_PER_W // CHUNK


def _emb_body(wte_hbm, ids_hbm, out_hbm, idx_v, buf, sem):
    wid = lax.axis_index("s") * NUM_CORES + lax.axis_index("c")
    base = wid * B_PER_W
    pltpu.sync_copy(ids_hbm.at[pl.ds(base, B_PER_W)], idx_v)
    for g in range(N_CHUNKS):
        off = g * CHUNK
        pltpu.async_copy(wte_hbm.at[idx_v.at[pl.ds(off, CHUNK)]], buf, sem).wait()
        pltpu.sync_copy(buf, out_hbm.at[pl.ds(base + off, CHUNK)])


@functools.partial(
    pl.kernel,
    out_type=jax.ShapeDtypeStruct((B_TOTAL, N_EMBD), jnp.float32),
    mesh=plsc.VectorSubcoreMesh(core_axis_name="c", subcore_axis_name="s"),
    scratch_types=[
        pltpu.VMEM((B_PER_W,), jnp.int32),
        pltpu.VMEM((CHUNK, N_EMBD), jnp.float32),
        pltpu.SemaphoreType.DMA,
    ],
)
def _emb(wte_hbm, ids_hbm, out_hbm, idx_v, buf, sem):
    _emb_body(wte_hbm, ids_hbm, out_hbm, idx_v, buf, sem)


def kernel(input_ids, wte):
    ids2 = input_ids.reshape(-1, input_ids.shape[-1])
    flat = ids2.reshape(-1).astype(jnp.int32)
    out = _emb(wte, flat)
    return out.reshape(ids2.shape + (N_EMBD,))


# speedup vs baseline: 1.6553x; 1.1714x over previous
import functools

import jax
import jax.numpy as jnp
from jax import lax
from jax.experimental import pallas as pl
from jax.experimental.pallas import tpu as pltpu
from jax.experimental.pallas import tpu_sc as plsc

VOCAB = 50304
N_EMBD = 1024
B_TOTAL = 4 * 4096
NUM_CORES = 2
NUM_SUBCORES = 16
NUM_WORKERS = NUM_CORES * NUM_SUBCORES
B_PER_W = B_TOTAL // NUM_WORKERS
CHUNK = 32
N_CHUNKS = B_PER_W // CHUNK


def _emb_body(wte_hbm, ids_hbm, out_hbm, idx_v, buf0, buf1, gsem, osem):
    wid = lax.axis_index("s") * NUM_CORES + lax.axis_index("c")
    base = wid * B_PER_W
    bufs = (buf0, buf1)
    pltpu.sync_copy(ids_hbm.at[pl.ds(base, B_PER_W)], idx_v)

    def gather(g):
        off = g * CHUNK
        return pltpu.async_copy(
            wte_hbm.at[idx_v.at[pl.ds(off, CHUNK)]], bufs[g % 2], gsem)

    def scatter(g):
        off = g * CHUNK
        return pltpu.async_copy(
            bufs[g % 2], out_hbm.at[pl.ds(base + off, CHUNK)], osem)

    gd = [None] * N_CHUNKS
    od = [None] * N_CHUNKS
    gd[0] = gather(0)
    for g in range(N_CHUNKS):
        if g + 1 < N_CHUNKS:
            if g >= 1:
                od[g - 1].wait()
            gd[g + 1] = gather(g + 1)
        gd[g].wait()
        od[g] = scatter(g)
    od[N_CHUNKS - 2].wait()
    od[N_CHUNKS - 1].wait()


@functools.partial(
    pl.kernel,
    out_type=jax.ShapeDtypeStruct((B_TOTAL, N_EMBD), jnp.float32),
    mesh=plsc.VectorSubcoreMesh(core_axis_name="c", subcore_axis_name="s"),
    scratch_types=[
        pltpu.VMEM((B_PER_W,), jnp.int32),
        pltpu.VMEM((CHUNK, N_EMBD), jnp.float32),
        pltpu.VMEM((CHUNK, N_EMBD), jnp.float32),
        pltpu.SemaphoreType.DMA,
        pltpu.SemaphoreType.DMA,
    ],
)
def _emb(wte_hbm, ids_hbm, out_hbm, idx_v, buf0, buf1, gsem, osem):
    _emb_body(wte_hbm, ids_hbm, out_hbm, idx_v, buf0, buf1, gsem, osem)


def kernel(input_ids, wte):
    ids2 = input_ids.reshape(-1, input_ids.shape[-1])
    flat = ids2.reshape(-1).astype(jnp.int32)
    out = _emb(wte, flat)
    return out.reshape(ids2.shape + (N_EMBD,))
